# baseline (speedup 1.0000x reference)
import math

import jax
import jax.numpy as jnp
from jax.experimental import pallas as pl


def _dot(a, b):
    return jnp.dot(a, b, preferred_element_type=jnp.float32,
                   precision=jax.lax.Precision.HIGHEST)


def _mlp_body(xin_ref, noise_ref,
              mw1, mb1, mg, mbe, mw2, mb2,
              sw1, sb1, sgg, sbe, sw2, sb2,
              z_ref):
    xm = xin_ref[...]

    def mlp(W1, b1, g, beta, W2, b2):
        h = _dot(xm, W1[...]) + b1[...]
        m = jnp.mean(h, axis=-1, keepdims=True)
        v = jnp.mean((h - m) ** 2, axis=-1, keepdims=True)
        h = (h - m) * jax.lax.rsqrt(v + 1e-5) * g[...] + beta[...]
        h = 0.5 * h * (1.0 + jax.lax.erf(h * (1.0 / math.sqrt(2.0))))
        return _dot(h, W2[...]) + b2[...]

    mu = mlp(mw1, mb1, mg, mbe, mw2, mb2)
    sg = mlp(sw1, sb1, sgg, sbe, sw2, sb2)
    z_ref[...] = mu + noise_ref[...] * sg


def kernel(x, mu_W1, mu_b1, mu_g, mu_beta, mu_W2, mu_b2,
           sg_W1, sg_b1, sg_g, sg_beta, sg_W2, sg_b2, noise, curr_indices):
    B, S, N = x.shape
    T = B * S
    K = curr_indices.shape[0]
    K2 = 2 * K
    H = mu_W1.shape[1]

    x_freq = jnp.fft.rfft(x, axis=-1)
    x_sel = jnp.take(x_freq, curr_indices, axis=-1)
    x_mlp_in = jnp.concatenate([jnp.real(x_sel), jnp.imag(x_sel)], axis=-1)

    xin2 = x_mlp_in.reshape(T, K2)
    noise2 = noise.reshape(T, K2)

    TT = 256
    grid = (T // TT,)

    def tok(i):
        return (i, 0)

    def rep(i):
        return (0, 0)

    full = lambda shape: pl.BlockSpec(shape, rep)

    z = pl.pallas_call(
        _mlp_body,
        grid=grid,
        in_specs=[
            pl.BlockSpec((TT, K2), tok),
            pl.BlockSpec((TT, K2), tok),
            full((K2, H)), full((1, H)), full((1, H)), full((1, H)),
            full((H, K2)), full((1, K2)),
            full((K2, H)), full((1, H)), full((1, H)), full((1, H)),
            full((H, K2)), full((1, K2)),
        ],
        out_specs=pl.BlockSpec((TT, K2), tok),
        out_shape=jax.ShapeDtypeStruct((T, K2), jnp.float32),
    )(xin2, noise2,
      mu_W1, mu_b1.reshape(1, H), mu_g.reshape(1, H), mu_beta.reshape(1, H),
      mu_W2, mu_b2.reshape(1, K2),
      sg_W1, sg_b1.reshape(1, H), sg_g.reshape(1, H), sg_beta.reshape(1, H),
      sg_W2, sg_b2.reshape(1, K2))

    z = z.reshape(B, S, K2)
    z_complex = jax.lax.complex(z[..., :K], z[..., K:])
    total = jnp.zeros(x_freq.shape, dtype=jnp.complex64)
    total = total.at[..., curr_indices].add(z_complex)
    out = jnp.fft.irfft(total, n=N, axis=-1)
    return out.astype(jnp.float32)


# speedup vs baseline: 1.0128x; 1.0128x over previous
import math

import jax
import jax.numpy as jnp
from jax.experimental import pallas as pl


def _dot(a, b):
    return jnp.dot(a, b, preferred_element_type=jnp.float32)


def _mlp_body(xin_ref, noise_ref,
              mw1, mb1, mg, mbe, mw2, mb2,
              sw1, sb1, sgg, sbe, sw2, sb2,
              z_ref):
    xm = xin_ref[...]

    def mlp(W1, b1, g, beta, W2, b2):
        h = _dot(xm, W1[...]) + b1[...]
        m = jnp.mean(h, axis=-1, keepdims=True)
        v = jnp.mean((h - m) ** 2, axis=-1, keepdims=True)
        h = (h - m) * jax.lax.rsqrt(v + 1e-5) * g[...] + beta[...]
        h = 0.5 * h * (1.0 + jax.lax.erf(h * (1.0 / math.sqrt(2.0))))
        return _dot(h, W2[...]) + b2[...]

    mu = mlp(mw1, mb1, mg, mbe, mw2, mb2)
    sg = mlp(sw1, sb1, sgg, sbe, sw2, sb2)
    z_ref[...] = mu + noise_ref[...] * sg


def kernel(x, mu_W1, mu_b1, mu_g, mu_beta, mu_W2, mu_b2,
           sg_W1, sg_b1, sg_g, sg_beta, sg_W2, sg_b2, noise, curr_indices):
    B, S, N = x.shape
    T = B * S
    K = curr_indices.shape[0]
    K2 = 2 * K
    H = mu_W1.shape[1]

    x_freq = jnp.fft.rfft(x, axis=-1)
    x_sel = jnp.take(x_freq, curr_indices, axis=-1)
    x_mlp_in = jnp.concatenate([jnp.real(x_sel), jnp.imag(x_sel)], axis=-1)

    xin2 = x_mlp_in.reshape(T, K2)
    noise2 = noise.reshape(T, K2)

    TT = 256
    grid = (T // TT,)

    def tok(i):
        return (i, 0)

    def rep(i):
        return (0, 0)

    full = lambda shape: pl.BlockSpec(shape, rep)

    z = pl.pallas_call(
        _mlp_body,
        grid=grid,
        in_specs=[
            pl.BlockSpec((TT, K2), tok),
            pl.BlockSpec((TT, K2), tok),
            full((K2, H)), full((1, H)), full((1, H)), full((1, H)),
            full((H, K2)), full((1, K2)),
            full((K2, H)), full((1, H)), full((1, H)), full((1, H)),
            full((H, K2)), full((1, K2)),
        ],
        out_specs=pl.BlockSpec((TT, K2), tok),
        out_shape=jax.ShapeDtypeStruct((T, K2), jnp.float32),
    )(xin2, noise2,
      mu_W1, mu_b1.reshape(1, H), mu_g.reshape(1, H), mu_beta.reshape(1, H),
      mu_W2, mu_b2.reshape(1, K2),
      sg_W1, sg_b1.reshape(1, H), sg_g.reshape(1, H), sg_beta.reshape(1, H),
      sg_W2, sg_b2.reshape(1, K2))

    z = z.reshape(B, S, K2)
    z_complex = jax.lax.complex(z[..., :K], z[..., K:])
    total = jnp.zeros(x_freq.shape, dtype=jnp.complex64)
    total = total.at[..., curr_indices].add(z_complex)
    out = jnp.fft.irfft(total, n=N, axis=-1)
    return out.astype(jnp.float32)


# speedup vs baseline: 1.0146x; 1.0018x over previous
import math

import jax
import jax.numpy as jnp
from jax.experimental import pallas as pl


def _dot(a, b):
    return jnp.dot(a, b, preferred_element_type=jnp.float32)


def _mlp_body(xin_ref, noise_ref,
              mw1, mb1, mg, mbe, mw2, mb2,
              sw1, sb1, sgg, sbe, sw2, sb2,
              z_ref):
    xm = xin_ref[...]

    def mlp(W1, b1, g, beta, W2, b2):
        h = _dot(xm, W1[...]) + b1[...]
        m = jnp.mean(h, axis=-1, keepdims=True)
        v = jnp.mean((h - m) ** 2, axis=-1, keepdims=True)
        h = (h - m) * jax.lax.rsqrt(v + 1e-5) * g[...] + beta[...]
        h = 0.5 * h * (1.0 + jax.lax.erf(h * (1.0 / math.sqrt(2.0))))
        return _dot(h, W2[...]) + b2[...]

    mu = mlp(mw1, mb1, mg, mbe, mw2, mb2)
    sg = mlp(sw1, sb1, sgg, sbe, sw2, sb2)
    z_ref[...] = mu + noise_ref[...] * sg


def kernel(x, mu_W1, mu_b1, mu_g, mu_beta, mu_W2, mu_b2,
           sg_W1, sg_b1, sg_g, sg_beta, sg_W2, sg_b2, noise, curr_indices):
    B, S, N = x.shape
    T = B * S
    K = curr_indices.shape[0]
    K2 = 2 * K
    H = mu_W1.shape[1]

    x_freq = jnp.fft.rfft(x, axis=-1)
    x_sel = jnp.take(x_freq, curr_indices, axis=-1)
    x_mlp_in = jnp.concatenate([jnp.real(x_sel), jnp.imag(x_sel)], axis=-1)

    xin2 = x_mlp_in.reshape(T, K2)
    noise2 = noise.reshape(T, K2)

    TT = 512
    grid = (T // TT,)

    def tok(i):
        return (i, 0)

    def rep(i):
        return (0, 0)

    full = lambda shape: pl.BlockSpec(shape, rep)

    z = pl.pallas_call(
        _mlp_body,
        grid=grid,
        in_specs=[
            pl.BlockSpec((TT, K2), tok),
            pl.BlockSpec((TT, K2), tok),
            full((K2, H)), full((1, H)), full((1, H)), full((1, H)),
            full((H, K2)), full((1, K2)),
            full((K2, H)), full((1, H)), full((1, H)), full((1, H)),
            full((H, K2)), full((1, K2)),
        ],
        out_specs=pl.BlockSpec((TT, K2), tok),
        out_shape=jax.ShapeDtypeStruct((T, K2), jnp.float32),
    )(xin2, noise2,
      mu_W1, mu_b1.reshape(1, H), mu_g.reshape(1, H), mu_beta.reshape(1, H),
      mu_W2, mu_b2.reshape(1, K2),
      sg_W1, sg_b1.reshape(1, H), sg_g.reshape(1, H), sg_beta.reshape(1, H),
      sg_W2, sg_b2.reshape(1, K2))

    z = z.reshape(B, S, K2)
    z_complex = jax.lax.complex(z[..., :K], z[..., K:])
    total = jnp.zeros(x_freq.shape, dtype=jnp.complex64)
    total = total.at[..., curr_indices].add(z_complex)
    out = jnp.fft.irfft(total, n=N, axis=-1)
    return out.astype(jnp.float32)
